# baseline (speedup 1.0000x reference)
import jax
import jax.numpy as jnp
from jax.experimental import pallas as pl

_R = 8


def _body(a_ref, wb_ref, otu_ref, out_ref):
    g = a_ref.shape[0]
    for j in range(g):
        m = jnp.concatenate([wb_ref[...], otu_ref[j]], axis=0)
        out_ref[:, 0, j, :] = jnp.dot(
            a_ref[j], m, preferred_element_type=jnp.float32
        )


def kernel(clr, otu_table, W_val, b_val):
    B, D = clr.shape
    E = otu_table.shape[1]
    R = _R
    nchunks = D // R
    G = 25
    nsteps = nchunks // G

    w = W_val[:, 0]
    w_diag = (jnp.eye(R, dtype=w.dtype)[:, :, None] * w[None, None, :]).reshape(R, R * E)
    b_tile = jnp.tile(b_val, R).reshape(1, R * E)
    wb = jnp.concatenate([w_diag, b_tile], axis=0)

    ones = jnp.ones((nchunks, B, 2), dtype=clr.dtype)
    a3 = jnp.concatenate(
        [clr.reshape(B, nchunks, R).transpose(1, 0, 2), ones], axis=2
    )
    otu3 = otu_table.reshape(nchunks, 1, R * E)

    out = pl.pallas_call(
        _body,
        grid=(nsteps,),
        in_specs=[
            pl.BlockSpec((G, B, R + 2), lambda s: (s, 0, 0)),
            pl.BlockSpec((R + 1, R * E), lambda s: (0, 0)),
            pl.BlockSpec((G, 1, R * E), lambda s: (s, 0, 0)),
        ],
        out_specs=pl.BlockSpec((B, 1, G, R * E), lambda s: (0, s, 0, 0)),
        out_shape=jax.ShapeDtypeStruct((B, nsteps, G, R * E), jnp.float32),
    )(a3, wb, otu3)
    return out.reshape(B, D, E)


# speedup vs baseline: 2.7299x; 2.7299x over previous
import jax
import jax.numpy as jnp
from jax.experimental import pallas as pl


def _fold_bias(otu_ref, b_ref, out_ref):
    out_ref[...] = otu_ref[...] + b_ref[...]


def _body(tpb_ref, clr_ref, w_ref, out_ref):
    b_idx = pl.program_id(1)
    blk = clr_ref[0]
    nb = blk.shape[1]
    onehot = (jax.lax.broadcasted_iota(jnp.int32, (1, nb), 1) == b_idx)
    col = jnp.sum(blk * onehot.astype(blk.dtype), axis=1, keepdims=True)
    out_ref[0] = tpb_ref[...] + col * w_ref[...]


def _pick_dblk(d: int) -> int:
    best = 8
    for cand in range(8, 1025, 8):
        if d % cand == 0:
            best = cand
    return best


def kernel(clr, otu_table, W_val, b_val):
    B, D = clr.shape
    E = otu_table.shape[1]
    dblk = _pick_dblk(D)
    ndb = D // dblk

    table_pb = pl.pallas_call(
        _fold_bias,
        out_shape=jax.ShapeDtypeStruct((D, E), jnp.float32),
    )(otu_table, b_val.reshape(1, E))

    clr3 = clr.T.reshape(ndb, dblk, B)
    w_row = W_val[:, 0].reshape(1, E)

    out = pl.pallas_call(
        _body,
        grid=(ndb, B),
        in_specs=[
            pl.BlockSpec((dblk, E), lambda d, b: (d, 0)),
            pl.BlockSpec((1, dblk, B), lambda d, b: (d, 0, 0)),
            pl.BlockSpec((1, E), lambda d, b: (0, 0)),
        ],
        out_specs=pl.BlockSpec((1, dblk, E), lambda d, b: (b, d, 0)),
        out_shape=jax.ShapeDtypeStruct((B, D, E), jnp.float32),
    )(table_pb, clr3, w_row)
    return out


# speedup vs baseline: 4.7134x; 1.7266x over previous
import jax
import jax.numpy as jnp
from jax.experimental import pallas as pl

_BBLK = 8


def _fold_bias(otu_ref, b_ref, out_ref):
    out_ref[...] = otu_ref[...] + b_ref[...]


def _body(tpb_ref, clr_ref, sel_ref, w_ref, out_ref):
    blk = clr_ref[0]
    cols = jnp.dot(blk, sel_ref[0], preferred_element_type=jnp.float32)
    for j in range(out_ref.shape[0]):
        out_ref[j] = tpb_ref[...] + cols[:, j:j + 1] * w_ref[...]


def _pick_dblk(d: int) -> int:
    best = 8
    for cand in range(8, 1025, 8):
        if d % cand == 0:
            best = cand
    return best


def kernel(clr, otu_table, W_val, b_val):
    B, D = clr.shape
    E = otu_table.shape[1]
    dblk = _pick_dblk(D)
    ndb = D // dblk
    bblk = _BBLK if B % _BBLK == 0 else 1
    nbb = B // bblk

    table_pb = pl.pallas_call(
        _fold_bias,
        out_shape=jax.ShapeDtypeStruct((D, E), jnp.float32),
    )(otu_table, b_val.reshape(1, E))

    clr3 = clr.T.reshape(ndb, dblk, B)
    w_row = W_val[:, 0].reshape(1, E)
    sel3 = (
        jax.lax.broadcasted_iota(jnp.int32, (nbb, B, bblk), 1)
        == jax.lax.broadcasted_iota(jnp.int32, (nbb, B, bblk), 2)
        + jax.lax.broadcasted_iota(jnp.int32, (nbb, B, bblk), 0) * bblk
    ).astype(jnp.float32)

    out = pl.pallas_call(
        _body,
        grid=(ndb, nbb),
        in_specs=[
            pl.BlockSpec((dblk, E), lambda d, g: (d, 0)),
            pl.BlockSpec((1, dblk, B), lambda d, g: (d, 0, 0)),
            pl.BlockSpec((1, B, bblk), lambda d, g: (g, 0, 0)),
            pl.BlockSpec((1, E), lambda d, g: (0, 0)),
        ],
        out_specs=pl.BlockSpec((bblk, dblk, E), lambda d, g: (g, d, 0)),
        out_shape=jax.ShapeDtypeStruct((B, D, E), jnp.float32),
    )(table_pb, clr3, sel3, w_row)
    return out


# speedup vs baseline: 4.7649x; 1.0109x over previous
import functools

import jax
import jax.numpy as jnp
from jax.experimental import pallas as pl
from jax.experimental.pallas import tpu as pltpu

_BBLK = 8
_NBUF = 4


def _fold_bias(otu_ref, b_ref, out_ref):
    out_ref[...] = otu_ref[...] + b_ref[...]


def _body(nbb, nsteps, tpb_ref, clr_ref, sel_ref, w_ref, out_ref, buf_ref, sems):
    i = pl.program_id(0)
    bblk = buf_ref.shape[1]
    dblk = buf_ref.shape[2]
    slot = jax.lax.rem(i, _NBUF)

    dst0 = out_ref.at[pl.ds(0, bblk), pl.ds(0, dblk), :]

    @pl.when(i >= _NBUF)
    def _wait_prev():
        pltpu.make_async_copy(buf_ref.at[slot], dst0, sems.at[slot]).wait()

    blk = clr_ref[0]
    cols = jnp.dot(
        blk, sel_ref[0],
        preferred_element_type=jnp.float32,
        precision=jax.lax.Precision.HIGHEST,
    )
    for j in range(bblk):
        buf_ref[slot, j] = tpb_ref[...] + cols[:, j:j + 1] * w_ref[...]

    d_idx = i // nbb
    g_idx = jax.lax.rem(i, nbb)
    dst = out_ref.at[pl.ds(g_idx * bblk, bblk), pl.ds(d_idx * dblk, dblk), :]
    copy = pltpu.make_async_copy(buf_ref.at[slot], dst, sems.at[slot])
    copy.start()

    @pl.when(i == nsteps - 1)
    def _drain():
        for k in range(_NBUF):
            pltpu.make_async_copy(buf_ref.at[k], dst0, sems.at[k]).wait()


def _pick_dblk(d: int) -> int:
    best = 8
    for cand in range(8, 1025, 8):
        if d % cand == 0:
            best = cand
    return best


def kernel(clr, otu_table, W_val, b_val):
    B, D = clr.shape
    E = otu_table.shape[1]
    dblk = _pick_dblk(D)
    ndb = D // dblk
    bblk = _BBLK if B % _BBLK == 0 else 1
    nbb = B // bblk
    nsteps = ndb * nbb

    table_pb = pl.pallas_call(
        _fold_bias,
        out_shape=jax.ShapeDtypeStruct((D, E), jnp.float32),
    )(otu_table, b_val.reshape(1, E))

    clr3 = clr.T.reshape(ndb, dblk, B)
    w_row = W_val[:, 0].reshape(1, E)
    sel3 = (
        jax.lax.broadcasted_iota(jnp.int32, (nbb, B, bblk), 1)
        == jax.lax.broadcasted_iota(jnp.int32, (nbb, B, bblk), 2)
        + jax.lax.broadcasted_iota(jnp.int32, (nbb, B, bblk), 0) * bblk
    ).astype(jnp.float32)

    out = pl.pallas_call(
        functools.partial(_body, nbb, nsteps),
        grid=(nsteps,),
        in_specs=[
            pl.BlockSpec((dblk, E), lambda i: (i // nbb, 0)),
            pl.BlockSpec((1, dblk, B), lambda i: (i // nbb, 0, 0)),
            pl.BlockSpec((1, B, bblk), lambda i: (i % nbb, 0, 0)),
            pl.BlockSpec((1, E), lambda i: (0, 0)),
        ],
        out_specs=pl.BlockSpec(memory_space=pltpu.MemorySpace.HBM),
        out_shape=jax.ShapeDtypeStruct((B, D, E), jnp.float32),
        scratch_shapes=[
            pltpu.VMEM((_NBUF, bblk, dblk, E), jnp.float32),
            pltpu.SemaphoreType.DMA((_NBUF,)),
        ],
    )(table_pb, clr3, sel3, w_row)
    return out
